# Initial kernel scaffold; baseline (speedup 1.0000x reference)
import functools

import jax
import jax.numpy as jnp
from jax import lax
from jax.experimental import pallas as pl
from jax.experimental.pallas import tpu as pltpu
from jax.experimental.pallas import tpu_sc as plsc

N_BUCKETS = 32
SCALE = 0.125

_THRESHOLDS = (1, 2, 3, 4, 5, 6, 7, 8, 9, 10, 11, 12, 13, 14, 15, 16,
               19, 21, 24, 27, 31, 35, 40, 46, 52, 59, 67, 77, 87, 99, 113)

_L = 16


def _make_sc_kernel(n_rows, n_cols):
    info = plsc.get_sparse_core_info()
    nc, ns = info.num_cores, info.num_subcores
    nw = nc * ns
    u_len = 2 * n_rows
    chunk = u_len // ns
    rows_per_w = n_rows // nw

    mesh = plsc.VectorSubcoreMesh(core_axis_name="c", subcore_axis_name="s")

    @functools.partial(
        pl.kernel,
        mesh=mesh,
        out_type=jax.ShapeDtypeStruct((n_rows, n_cols), jnp.float32),
        scratch_types=[
            pltpu.VMEM((N_BUCKETS,), jnp.float32),
            pltpu.VMEM((u_len,), jnp.float32),
            pltpu.VMEM_SHARED((u_len,), jnp.float32),
            pltpu.SemaphoreType.DMA,
        ],
    )
    def k(table_hbm, out_hbm, table_v, u_full, u_shared, sem):
        cid = lax.axis_index("c")
        sid = lax.axis_index("s")
        wid = sid * nc + cid

        pltpu.sync_copy(table_hbm, table_v)

        base = sid * chunk
        for v in range(chunk // _L):
            p = base + v * _L + lax.iota(jnp.int32, _L)
            n = jnp.maximum((n_rows - 1) - p, 0)
            bkt = jnp.zeros((_L,), jnp.int32)
            for t in _THRESHOLDS:
                bkt = bkt + (n >= t).astype(jnp.int32)
            vals = plsc.load_gather(table_v, [bkt]) * SCALE
            u_full[pl.ds(base + v * _L, _L)] = vals

        pltpu.sync_copy(u_full.at[pl.ds(base, chunk)],
                        u_shared.at[pl.ds(base, chunk)])
        plsc.subcore_barrier()
        pltpu.sync_copy(u_shared, u_full)

        i0 = wid * rows_per_w
        copies = []
        for r in range(rows_per_w):
            i = i0 + r
            start = (n_rows - 1) - i
            copies.append(
                pltpu.async_copy(u_full.at[pl.ds(start, n_cols)],
                                 out_hbm.at[i], sem))
        for c in copies:
            c.wait()

    return k


def kernel(x, table):
    i, j = x.shape[-2], x.shape[-1]
    return _make_sc_kernel(i, j)(table.reshape(-1))


# speedup vs baseline: 1.8202x; 1.8202x over previous
import functools

import jax
import jax.numpy as jnp
from jax import lax
from jax.experimental import pallas as pl
from jax.experimental.pallas import tpu as pltpu
from jax.experimental.pallas import tpu_sc as plsc

N_BUCKETS = 32
SCALE = 0.125

_THRESHOLDS = (1, 2, 3, 4, 5, 6, 7, 8, 9, 10, 11, 12, 13, 14, 15, 16,
               19, 21, 24, 27, 31, 35, 40, 46, 52, 59, 67, 77, 87, 99, 113)

_L = 16


def _make_sc_kernel(n_rows, n_cols):
    info = plsc.get_sparse_core_info()
    nc, ns = info.num_cores, info.num_subcores
    nw = nc * ns
    u_len = 2 * n_rows
    chunk = u_len // ns
    rows_per_w = n_rows // nw

    mesh = plsc.VectorSubcoreMesh(core_axis_name="c", subcore_axis_name="s")

    @functools.partial(
        pl.kernel,
        mesh=mesh,
        out_type=jax.ShapeDtypeStruct((n_rows, n_cols), jnp.float32),
        compiler_params=pltpu.CompilerParams(needs_layout_passes=False,
                                             use_tc_tiling_on_sc=False),
        scratch_types=[
            pltpu.VMEM((N_BUCKETS,), jnp.float32),
            pltpu.VMEM((u_len,), jnp.float32),
            pltpu.VMEM((u_len,), jnp.float32),
            pltpu.VMEM_SHARED((u_len,), jnp.float32),
            pltpu.SemaphoreType.DMA,
        ],
    )
    def k(table_hbm, out_hbm, table_v, u_full, u_stag, u_shared, sem):
        cid = lax.axis_index("c")
        sid = lax.axis_index("s")
        wid = sid * nc + cid
        res = wid % 8
        blk = wid // 8

        pltpu.sync_copy(table_hbm, table_v)

        base = sid * chunk
        for v in range(chunk // _L):
            p = base + v * _L + lax.iota(jnp.int32, _L)
            n = jnp.maximum((n_rows - 1) - p, 0)
            bkt = jnp.zeros((_L,), jnp.int32)
            for t in _THRESHOLDS:
                bkt = bkt + jnp.minimum(jnp.maximum(n - (t - 1), 0), 1)
            vals = plsc.load_gather(table_v, [bkt]) * SCALE
            u_full[pl.ds(base + v * _L, _L)] = vals

        pltpu.sync_copy(u_full.at[pl.ds(base, chunk)],
                        u_shared.at[pl.ds(base, chunk)])
        plsc.subcore_barrier()
        pltpu.sync_copy(u_shared, u_full)

        def shift_body(vv, _):
            idx = jnp.minimum(vv * _L + res + lax.iota(jnp.int32, _L),
                              u_len - 1)
            u_stag[pl.ds(vv * _L, _L)] = plsc.load_gather(u_full, [idx])
            return _
        lax.fori_loop(0, u_len // _L, shift_body, 0)

        copies = []
        for r in range(rows_per_w):
            step = blk * rows_per_w + r
            i = (7 - res) + 8 * step
            q = (n_rows - 8) - 8 * step
            copies.append(
                pltpu.async_copy(u_stag.at[pl.ds(q, n_cols)],
                                 out_hbm.at[i], sem))
        for c in copies:
            c.wait()

    return k


def kernel(x, table):
    i, j = x.shape[-2], x.shape[-1]
    return _make_sc_kernel(i, j)(table.reshape(-1))


# speedup vs baseline: 1.8273x; 1.0039x over previous
import functools

import jax
import jax.numpy as jnp
from jax import lax
from jax.experimental import pallas as pl
from jax.experimental.pallas import tpu as pltpu
from jax.experimental.pallas import tpu_sc as plsc

N_BUCKETS = 32
SCALE = 0.125

_THRESHOLDS = (1, 2, 3, 4, 5, 6, 7, 8, 9, 10, 11, 12, 13, 14, 15, 16,
               19, 21, 24, 27, 31, 35, 40, 46, 52, 59, 67, 77, 87, 99, 113)

_L = 16


def _make_sc_kernel(n_rows, n_cols):
    info = plsc.get_sparse_core_info()
    nc, ns = info.num_cores, info.num_subcores
    nw = nc * ns
    u_len = 2 * n_rows
    chunk = u_len // ns
    rows_per_w = n_rows // nw

    mesh = plsc.VectorSubcoreMesh(core_axis_name="c", subcore_axis_name="s")

    @functools.partial(
        pl.kernel,
        mesh=mesh,
        out_type=jax.ShapeDtypeStruct((n_rows, n_cols), jnp.float32),
        compiler_params=pltpu.CompilerParams(needs_layout_passes=False,
                                             use_tc_tiling_on_sc=False),
        scratch_types=[
            pltpu.VMEM((N_BUCKETS,), jnp.float32),
            pltpu.VMEM((u_len,), jnp.float32),
            pltpu.VMEM((u_len,), jnp.float32),
            pltpu.VMEM_SHARED((u_len,), jnp.float32),
            pltpu.SemaphoreType.DMA,
        ],
    )
    def k(table_hbm, out_hbm, table_v, u_full, u_stag, u_shared, sem):
        cid = lax.axis_index("c")
        sid = lax.axis_index("s")
        wid = sid * nc + cid
        res = wid % 16
        blk = wid // 16

        pltpu.sync_copy(table_hbm, table_v)

        base = sid * chunk
        for v in range(chunk // _L):
            p = base + v * _L + lax.iota(jnp.int32, _L)
            n = jnp.maximum((n_rows - 1) - p, 0)
            bkt = jnp.zeros((_L,), jnp.int32)
            for t in _THRESHOLDS:
                bkt = bkt + jnp.minimum(jnp.maximum(n - (t - 1), 0), 1)
            vals = plsc.load_gather(table_v, [bkt]) * SCALE
            u_full[pl.ds(base + v * _L, _L)] = vals

        pltpu.sync_copy(u_full.at[pl.ds(base, chunk)],
                        u_shared.at[pl.ds(base, chunk)])
        plsc.subcore_barrier()
        pltpu.sync_copy(u_shared, u_full)

        def shift_body(vv, _):
            idx = jnp.minimum(vv * _L + res + lax.iota(jnp.int32, _L),
                              u_len - 1)
            u_stag[pl.ds(vv * _L, _L)] = plsc.load_gather(u_full, [idx])
            return _
        lax.fori_loop(0, u_len // _L, shift_body, 0)

        copies = []
        for r in range(rows_per_w):
            step = blk * rows_per_w + r
            i = (15 - res) + 16 * step
            q = (n_rows - 16) - 16 * step
            copies.append(
                pltpu.async_copy(u_stag.at[pl.ds(q, n_cols)],
                                 out_hbm.at[i], sem))
        for c in copies:
            c.wait()

    return k


def kernel(x, table):
    i, j = x.shape[-2], x.shape[-1]
    return _make_sc_kernel(i, j)(table.reshape(-1))
